# baseline (speedup 1.0000x reference)
import functools

import jax
import jax.numpy as jnp
from jax import lax
from jax.experimental import pallas as pl
from jax.experimental.pallas import tpu as pltpu
from jax.experimental.pallas import tpu_sc as plsc

N = 10000
E = 320000
NC, NS = 2, 16
NW = NC * NS
K = 128
EPAD = 327680
NJUNK = 8
NBD = EPAD // NW // K
NBS = EPAD // NS // K
NBUF = {64: 5, 32: 5}
TPB = N // NS

_SC_PARAMS = dict(compiler_params=pltpu.CompilerParams(use_tc_tiling_on_sc=False))


def _mesh():
    return plsc.VectorSubcoreMesh(core_axis_name="c", subcore_axis_name="s")


@functools.cache
def _get_sc_degree():
    @functools.partial(
        pl.kernel,
        mesh=_mesh(),
        out_type=jax.ShapeDtypeStruct((NC, N, 16), jnp.float32),
        scratch_types=[
            pltpu.VMEM((NBD, K), jnp.int32),
            pltpu.VMEM((K, 16), jnp.float32),
            pltpu.VMEM_SHARED((N + NJUNK, 16), jnp.float32),
        ],
        **_SC_PARAMS,
    )
    def _sc_degree(dst_hbm, ones_hbm, zeros_hbm, out_hbm, dstv, onesv, acc):
        c = lax.axis_index("c")
        s = lax.axis_index("s")
        wid = s * NC + c
        pltpu.sync_copy(zeros_hbm.at[pl.ds(s * TPB, TPB)], acc.at[pl.ds(s * TPB, TPB)])
        pltpu.sync_copy(dst_hbm.at[wid], dstv)
        pltpu.sync_copy(ones_hbm, onesv)
        plsc.subcore_barrier()

        def body(i, carry):
            pltpu.sync_copy(onesv, acc.at[dstv.at[i]], add=True)
            return carry

        lax.fori_loop(0, NBD, body, 0)
        plsc.subcore_barrier()
        pltpu.sync_copy(acc.at[pl.ds(s * TPB, TPB)], out_hbm.at[c, pl.ds(s * TPB, TPB)])

    return _sc_degree


@functools.cache
def _make_sc_segsum(Dh):
    nbuf = NBUF[Dh]

    @functools.partial(
        pl.kernel,
        mesh=_mesh(),
        out_type=jax.ShapeDtypeStruct((NC, N, Dh), jnp.float32),
        scratch_types=[
            pltpu.VMEM((NBS, K), jnp.int32),
            pltpu.VMEM((NBS, K), jnp.int32),
            [pltpu.VMEM((K, Dh), jnp.float32) for _ in range(nbuf)],
            pltpu.VMEM_SHARED((N + NJUNK, Dh), jnp.float32),
            pltpu.SemaphoreType.DMA,
        ],
        **_SC_PARAMS,
    )
    def _sc_segsum(y_hbm, src_hbm, dst_hbm, out_hbm,
                   srcv, dstv, bufs, acc, sem):
        c = lax.axis_index("c")
        s = lax.axis_index("s")
        pltpu.sync_copy(src_hbm.at[s], srcv)
        pltpu.sync_copy(dst_hbm.at[s], dstv)

        def run(cc):
            tbl = y_hbm.at[cc]
            pltpu.sync_copy(tbl.at[pl.ds(s * TPB, TPB)], acc.at[pl.ds(s * TPB, TPB)])
            plsc.subcore_barrier()
            for b in range(nbuf):
                pltpu.async_copy(tbl.at[srcv.at[b]], bufs[b], sem)

            def body(j, carry):
                for b in range(nbuf):
                    i = j * nbuf + b
                    pltpu.make_async_copy(tbl.at[srcv.at[i]], bufs[b], sem).wait()
                    pltpu.sync_copy(bufs[b], acc.at[dstv.at[i]], add=True)

                    @pl.when(i + nbuf < NBS)
                    def _():
                        pltpu.async_copy(tbl.at[srcv.at[i + nbuf]], bufs[b], sem)

                return carry

            lax.fori_loop(0, NBS // nbuf, body, 0)

        @pl.when(c == 0)
        def _():
            run(0)

        @pl.when(c == 1)
        def _():
            run(1)

        plsc.subcore_barrier()
        pltpu.sync_copy(acc.at[pl.ds(s * TPB, TPB)],
                        out_hbm.at[c, pl.ds(s * TPB, TPB)])

    return _sc_segsum


@functools.cache
def _get_sc_segsum_final():
    Dh = 32
    nbuf = NBUF[Dh]

    @functools.partial(
        pl.kernel,
        mesh=_mesh(),
        out_type=jax.ShapeDtypeStruct((2, N, Dh), jnp.float32),
        scratch_types=[
            pltpu.VMEM((NBS, K), jnp.int32),
            pltpu.VMEM((NBS, K), jnp.int32),
            [pltpu.VMEM((K, Dh), jnp.float32) for _ in range(nbuf)],
            pltpu.VMEM((TPB, Dh), jnp.float32),
            pltpu.VMEM((TPB, Dh), jnp.float32),
            pltpu.VMEM((Dh,), jnp.float32),
            pltpu.VMEM_SHARED((N + NJUNK, Dh), jnp.float32),
            pltpu.SemaphoreType.DMA,
        ],
        **_SC_PARAMS,
    )
    def _sc_segsum_final(y_hbm, src_hbm, dst_hbm, dv_hbm, b2_hbm, out_hbm,
                         srcv, dstv, bufs, accT, dvT, b2T, acc, sem):
        c = lax.axis_index("c")
        s = lax.axis_index("s")
        pltpu.sync_copy(src_hbm.at[s], srcv)
        pltpu.sync_copy(dst_hbm.at[s], dstv)

        def run(cc):
            tbl = y_hbm.at[cc]
            pltpu.sync_copy(tbl.at[pl.ds(s * TPB, TPB)], acc.at[pl.ds(s * TPB, TPB)])
            plsc.subcore_barrier()
            for b in range(nbuf):
                pltpu.async_copy(tbl.at[srcv.at[b]], bufs[b], sem)

            def body(j, carry):
                for b in range(nbuf):
                    i = j * nbuf + b
                    pltpu.make_async_copy(tbl.at[srcv.at[i]], bufs[b], sem).wait()
                    pltpu.sync_copy(bufs[b], acc.at[dstv.at[i]], add=True)

                    @pl.when(i + nbuf < NBS)
                    def _():
                        pltpu.async_copy(tbl.at[srcv.at[i + nbuf]], bufs[b], sem)

                return carry

            lax.fori_loop(0, NBS // nbuf, body, 0)

        @pl.when(c == 0)
        def _():
            run(0)

        @pl.when(c == 1)
        def _():
            run(1)

        plsc.subcore_barrier()
        pltpu.sync_copy(acc.at[pl.ds(s * TPB, TPB)], accT)
        pltpu.sync_copy(dv_hbm.at[pl.ds(s * TPB, TPB)], dvT)
        pltpu.sync_copy(b2_hbm.at[c], b2T)

        def fin(r, carry):
            for j in range(Dh // 16):
                sl = pl.ds(j * 16, 16)
                accT[r, sl] = accT[r, sl] * dvT[r, sl] + b2T[sl]
            return carry

        lax.fori_loop(0, TPB, fin, 0)

        pltpu.sync_copy(accT, out_hbm.at[c, pl.ds(s * TPB, TPB)])

    return _sc_segsum_final


_BR = 1000


def _dinv_block(d0_ref, d1_ref):
    deg = 1.0 + d0_ref[:, 0:1] + d1_ref[:, 0:1]
    return lax.rsqrt(deg)


def _tc_y1(x, W1, d0, d1):
    F, H = W1.shape

    def body(x_ref, w_ref, d0_ref, d1_ref, y_ref):
        dinv = _dinv_block(d0_ref, d1_ref)
        yblk = jnp.dot(x_ref[...], w_ref[...],
                       preferred_element_type=jnp.float32) * dinv
        y_ref[0, :, :] = yblk[:, : H // 2]
        y_ref[1, :, :] = yblk[:, H // 2:]

    return pl.pallas_call(
        body,
        grid=(N // _BR,),
        in_specs=[
            pl.BlockSpec((_BR, F), lambda i: (i, 0)),
            pl.BlockSpec((F, H), lambda i: (0, 0)),
            pl.BlockSpec((_BR, 16), lambda i: (i, 0)),
            pl.BlockSpec((_BR, 16), lambda i: (i, 0)),
        ],
        out_specs=pl.BlockSpec((2, _BR, H // 2), lambda i: (0, i, 0)),
        out_shape=jax.ShapeDtypeStruct((2, N, H // 2), jnp.float32),
    )(x, W1, d0, d1)


def _tc_mid(a, d0, d1, b1, W2):
    H, C = W2.shape

    def body(a_ref, d0_ref, d1_ref, b1_ref, w2_ref, y2_ref, dv_ref):
        dinv = _dinv_block(d0_ref, d1_ref)
        ssum = jnp.concatenate([a_ref[0], a_ref[1]], axis=-1)
        h = jnp.maximum(dinv * ssum + b1_ref[...], 0.0)
        yy = jnp.dot(h, w2_ref[...], preferred_element_type=jnp.float32) * dinv
        y2_ref[0, :, :] = yy[:, : C // 2]
        y2_ref[1, :, :] = yy[:, C // 2:]
        dv_ref[...] = jnp.broadcast_to(dinv, (dinv.shape[0], C // 2))

    return pl.pallas_call(
        body,
        grid=(N // _BR,),
        in_specs=[
            pl.BlockSpec((2, _BR, H // 2), lambda i: (0, i, 0)),
            pl.BlockSpec((_BR, 16), lambda i: (i, 0)),
            pl.BlockSpec((_BR, 16), lambda i: (i, 0)),
            pl.BlockSpec((1, H), lambda i: (0, 0)),
            pl.BlockSpec((H, C), lambda i: (0, 0)),
        ],
        out_specs=[
            pl.BlockSpec((2, _BR, C // 2), lambda i: (0, i, 0)),
            pl.BlockSpec((_BR, C // 2), lambda i: (i, 0)),
        ],
        out_shape=[
            jax.ShapeDtypeStruct((2, N, C // 2), jnp.float32),
            jax.ShapeDtypeStruct((N, C // 2), jnp.float32),
        ],
    )(a, d0, d1, b1, W2)


def kernel(inputs, edge_index, W1, b1, W2, b2):
    npad = EPAD - E
    src32 = jnp.concatenate(
        [edge_index[0].astype(jnp.int32), jnp.zeros((npad,), jnp.int32)])
    dst32 = jnp.concatenate(
        [edge_index[1].astype(jnp.int32),
         N + (jnp.arange(npad, dtype=jnp.int32) % NJUNK)])
    src_s = src32.reshape(NS, NBS, K)
    dst_s = dst32.reshape(NS, NBS, K)
    dst_w = dst32.reshape(NW, NBD, K)
    ones16 = jnp.ones((K, 16), jnp.float32)
    zeros16 = jnp.zeros((N, 16), jnp.float32)

    dega = _get_sc_degree()(dst_w, ones16, zeros16)
    d0, d1 = dega[0], dega[1]

    y1 = _tc_y1(inputs, W1, d0, d1)
    acc1 = _make_sc_segsum(64)(y1, src_s, dst_s)
    y2, dv = _tc_mid(acc1, d0, d1, b1.reshape(1, -1), W2)
    oh = _get_sc_segsum_final()(y2, src_s, dst_s, dv, b2.reshape(2, 32))
    return jnp.concatenate([oh[0], oh[1]], axis=1)
